# baseline (speedup 1.0000x reference)
import functools

import jax
import jax.numpy as jnp
from jax import lax
from jax.experimental import pallas as pl
from jax.experimental.pallas import tpu as pltpu
from jax.experimental.pallas import tpu_sc as plsc

B = 4
S = 2048
D = 1024
N = B * S
NC = 2
NS = 16
NW = NC * NS
S_PER_W = S // NW
CS = 8
NCH = S_PER_W // CS
NBUF = 3
PREF = 2
LANES = 16
COLS = D // LANES
SCALE = 32.0

_mesh = plsc.VectorSubcoreMesh(
    core_axis_name="c", subcore_axis_name="s", num_cores=NC, num_subcores=NS
)


@functools.partial(
    pl.kernel,
    out_type=jax.ShapeDtypeStruct((B, S, D), jnp.float32),
    mesh=_mesh,
    scratch_types=[
        pltpu.VMEM((B, S_PER_W), jnp.int32),
        pltpu.VMEM((NBUF, B, CS, D), jnp.float32),
        pltpu.VMEM((NBUF, CS, D), jnp.float32),
        pltpu.SemaphoreType.DMA,
        pltpu.SemaphoreType.DMA,
        pltpu.SemaphoreType.DMA,
        pltpu.SemaphoreType.DMA,
        pltpu.SemaphoreType.DMA,
        pltpu.SemaphoreType.DMA,
    ],
)
def _emb_kernel(x_hbm, table_hbm, pos_hbm, out_hbm,
                idx_v, rows_v, pos_v, g0, g1, g2, o0, o1, o2):
    gsem = (g0, g1, g2)
    osem = (o0, o1, o2)
    wid = lax.axis_index("s") * NC + lax.axis_index("c")
    s0 = wid * S_PER_W

    for b in range(B):
        pltpu.sync_copy(x_hbm.at[b, pl.ds(s0, S_PER_W)], idx_v.at[b])

    def issue_gather(c, t):
        descs = [
            pltpu.async_copy(
                table_hbm.at[idx_v.at[b, pl.ds(c * CS, CS)]],
                rows_v.at[t, b], gsem[t])
            for b in range(B)
        ]
        descs.append(
            pltpu.async_copy(pos_hbm.at[pl.ds(s0 + c * CS, CS)],
                             pos_v.at[t], gsem[t]))
        return descs

    def issue_out(c, t):
        return [
            pltpu.async_copy(
                rows_v.at[t, b],
                out_hbm.at[b, pl.ds(s0 + c * CS, CS)], osem[t])
            for b in range(B)
        ]

    def compute(t):
        @plsc.parallel_loop(0, CS * COLS, unroll=2)
        def _(j):
            r = j >> 6
            col = (j & (COLS - 1)) * LANES
            p = pos_v[t, r, pl.ds(col, LANES)]
            for b in range(B):
                v = rows_v[t, b, r, pl.ds(col, LANES)]
                rows_v[t, b, r, pl.ds(col, LANES)] = v * SCALE + p

    pend_g = [None] * NBUF
    pend_o = [None] * NBUF
    for k in range(min(PREF, NCH)):
        pend_g[k % NBUF] = issue_gather(k, k % NBUF)
    for c in range(NCH):
        t = c % NBUF
        if c + PREF < NCH:
            tn = (c + PREF) % NBUF
            if pend_o[tn] is not None:
                for dsc in pend_o[tn]:
                    dsc.wait()
                pend_o[tn] = None
            pend_g[tn] = issue_gather(c + PREF, tn)
        for dsc in pend_g[t]:
            dsc.wait()
        if pend_o[t] is not None:
            for dsc in pend_o[t]:
                dsc.wait()
            pend_o[t] = None
        compute(t)
        pend_o[t] = issue_out(c, t)
    for t in range(NBUF):
        if pend_o[t] is not None:
            for dsc in pend_o[t]:
                dsc.wait()


def kernel(x, table, pos_encoding):
    return _emb_kernel(x.astype(jnp.int32), table, pos_encoding)


# speedup vs baseline: 1.0270x; 1.0270x over previous
import functools

import jax
import jax.numpy as jnp
from jax import lax
from jax.experimental import pallas as pl
from jax.experimental.pallas import tpu as pltpu
from jax.experimental.pallas import tpu_sc as plsc

B = 4
S = 2048
D = 1024
N = B * S
NC = 2
NS = 16
NW = NC * NS
S_PER_W = S // NW
CS = 8
NCH = S_PER_W // CS
NBUF = 3
PREF = 2
LANES = 16
COLS = D // LANES
SCALE = 32.0

_mesh = plsc.VectorSubcoreMesh(
    core_axis_name="c", subcore_axis_name="s", num_cores=NC, num_subcores=NS
)


@functools.partial(
    pl.kernel,
    out_type=jax.ShapeDtypeStruct((B, S, D), jnp.float32),
    mesh=_mesh,
    scratch_types=[
        pltpu.VMEM((B, S_PER_W), jnp.int32),
        pltpu.VMEM((NBUF, B, CS, D), jnp.float32),
        pltpu.VMEM((NBUF, CS, D), jnp.float32),
        pltpu.SemaphoreType.DMA,
        pltpu.SemaphoreType.DMA,
        pltpu.SemaphoreType.DMA,
        pltpu.SemaphoreType.DMA,
        pltpu.SemaphoreType.DMA,
        pltpu.SemaphoreType.DMA,
    ],
)
def _emb_kernel(x_hbm, table_hbm, pos_hbm, out_hbm,
                idx_v, rows_v, pos_v, g0, g1, g2, o0, o1, o2):
    gsem = (g0, g1, g2)
    osem = (o0, o1, o2)
    wid = lax.axis_index("s") * NC + lax.axis_index("c")
    s0 = wid * S_PER_W

    idescs = [
        pltpu.async_copy(x_hbm.at[b, pl.ds(s0, S_PER_W)], idx_v.at[b], g0)
        for b in range(B)
    ]
    for dsc in idescs:
        dsc.wait()

    def issue_gather(c, t):
        descs = [
            pltpu.async_copy(
                table_hbm.at[idx_v.at[b, pl.ds(c * CS, CS)]],
                rows_v.at[t, b], gsem[t])
            for b in range(B)
        ]
        descs.append(
            pltpu.async_copy(pos_hbm.at[pl.ds(s0 + c * CS, CS)],
                             pos_v.at[t], gsem[t]))
        return descs

    def issue_out(c, t):
        return [
            pltpu.async_copy(
                rows_v.at[t, b],
                out_hbm.at[b, pl.ds(s0 + c * CS, CS)], osem[t])
            for b in range(B)
        ]

    def compute(t):
        @plsc.parallel_loop(0, CS * COLS, unroll=2)
        def _(j):
            r = j >> 6
            col = (j & (COLS - 1)) * LANES
            p = pos_v[t, r, pl.ds(col, LANES)]
            for b in range(B):
                v = rows_v[t, b, r, pl.ds(col, LANES)]
                rows_v[t, b, r, pl.ds(col, LANES)] = v * SCALE + p

    pend_g = [None] * NBUF
    pend_o = [None] * NBUF
    for k in range(min(PREF, NCH)):
        pend_g[k % NBUF] = issue_gather(k, k % NBUF)
    for c in range(NCH):
        t = c % NBUF
        if c + PREF < NCH:
            tn = (c + PREF) % NBUF
            if pend_o[tn] is not None:
                for dsc in pend_o[tn]:
                    dsc.wait()
                pend_o[tn] = None
            pend_g[tn] = issue_gather(c + PREF, tn)
        for dsc in pend_g[t]:
            dsc.wait()
        if pend_o[t] is not None:
            for dsc in pend_o[t]:
                dsc.wait()
            pend_o[t] = None
        compute(t)
        pend_o[t] = issue_out(c, t)
    for t in range(NBUF):
        if pend_o[t] is not None:
            for dsc in pend_o[t]:
                dsc.wait()


def kernel(x, table, pos_encoding):
    return _emb_kernel(x.astype(jnp.int32), table, pos_encoding)


# speedup vs baseline: 1.0333x; 1.0061x over previous
import functools

import jax
import jax.numpy as jnp
from jax import lax
from jax.experimental import pallas as pl
from jax.experimental.pallas import tpu as pltpu
from jax.experimental.pallas import tpu_sc as plsc

B = 4
S = 2048
D = 1024
N = B * S
NC = 2
NS = 16
NW = NC * NS
S_PER_W = S // NW
CS = 8
NCH = S_PER_W // CS
NBUF = 3
PREF = 2
LANES = 16
COLS = D // LANES
SCALE = 32.0

_mesh = plsc.VectorSubcoreMesh(
    core_axis_name="c", subcore_axis_name="s", num_cores=NC, num_subcores=NS
)


@functools.partial(
    pl.kernel,
    out_type=jax.ShapeDtypeStruct((B, S, D), jnp.float32),
    mesh=_mesh,
    scratch_types=[
        pltpu.VMEM((B, S_PER_W), jnp.int32),
        pltpu.VMEM((NBUF, B, CS, D), jnp.float32),
        pltpu.VMEM((NBUF, CS, D), jnp.float32),
        pltpu.SemaphoreType.DMA,
        pltpu.SemaphoreType.DMA,
        pltpu.SemaphoreType.DMA,
        pltpu.SemaphoreType.DMA,
        pltpu.SemaphoreType.DMA,
        pltpu.SemaphoreType.DMA,
    ],
)
def _emb_kernel(x_hbm, table_hbm, pos_hbm, out_hbm,
                idx_v, rows_v, pos_v, g0, g1, g2, o0, o1, o2):
    gsem = (g0, g1, g2)
    osem = (o0, o1, o2)
    wid = lax.axis_index("s") * NC + lax.axis_index("c")
    s0 = wid * S_PER_W

    def issue_pos(c, t):
        return pltpu.async_copy(pos_hbm.at[pl.ds(s0 + c * CS, CS)],
                                pos_v.at[t], gsem[t])

    def issue_rows(c, t):
        return [
            pltpu.async_copy(
                table_hbm.at[idx_v.at[b, pl.ds(c * CS, CS)]],
                rows_v.at[t, b], gsem[t])
            for b in range(B)
        ]

    def issue_gather(c, t):
        descs = issue_rows(c, t)
        descs.append(issue_pos(c, t))
        return descs

    idescs = [
        pltpu.async_copy(x_hbm.at[b, pl.ds(s0, S_PER_W)], idx_v.at[b], o0)
        for b in range(B)
    ]
    pos_primed = [issue_pos(k, k % NBUF) for k in range(min(PREF, NCH))]
    for dsc in idescs:
        dsc.wait()

    def issue_out(c, t):
        return [
            pltpu.async_copy(
                rows_v.at[t, b],
                out_hbm.at[b, pl.ds(s0 + c * CS, CS)], osem[t])
            for b in range(B)
        ]

    def compute(t):
        @plsc.parallel_loop(0, CS * COLS, unroll=2)
        def _(j):
            r = j >> 6
            col = (j & (COLS - 1)) * LANES
            p = pos_v[t, r, pl.ds(col, LANES)]
            for b in range(B):
                v = rows_v[t, b, r, pl.ds(col, LANES)]
                rows_v[t, b, r, pl.ds(col, LANES)] = v * SCALE + p

    pend_g = [None] * NBUF
    pend_o = [None] * NBUF
    for k in range(min(PREF, NCH)):
        pend_g[k % NBUF] = issue_rows(k, k % NBUF) + [pos_primed[k]]
    for c in range(NCH):
        t = c % NBUF
        if c + PREF < NCH:
            tn = (c + PREF) % NBUF
            if pend_o[tn] is not None:
                for dsc in pend_o[tn]:
                    dsc.wait()
                pend_o[tn] = None
            pend_g[tn] = issue_gather(c + PREF, tn)
        for dsc in pend_g[t]:
            dsc.wait()
        if pend_o[t] is not None:
            for dsc in pend_o[t]:
                dsc.wait()
            pend_o[t] = None
        compute(t)
        pend_o[t] = issue_out(c, t)
    for t in range(NBUF):
        if pend_o[t] is not None:
            for dsc in pend_o[t]:
                dsc.wait()


def kernel(x, table, pos_encoding):
    return _emb_kernel(x.astype(jnp.int32), table, pos_encoding)
